# baseline (speedup 1.0000x reference)
import functools

import jax
import jax.numpy as jnp
from jax import lax
from jax.experimental import pallas as pl
from jax.experimental.pallas import tpu as pltpu
from jax.experimental.pallas import tpu_sc as plsc

B = 16384
C = 1000
SMOOTH = 0.1
EPS = 1e-12

NSPLIT = 8
STEP_ROWS = 256
GRID = B // (STEP_ROWS * NSPLIT)
HALF = STEP_ROWS // 2


def _dense_body(*refs):
    x_refs = refs[:NSPLIT]
    l_refs = refs[NSPLIT:2 * NSPLIT]
    o_ref, acc_ref = refs[2 * NSPLIT], refs[2 * NSPLIT + 1]
    step = pl.program_id(0)

    @pl.when(step == 0)
    def _():
        acc_ref[0, 0] = 0.0

    s = 0.0
    for x_ref, l_ref in zip(x_refs, l_refs):
        x = x_ref[...]
        cols = lax.broadcasted_iota(jnp.int32, (STEP_ROWS, C), 1)
        m = cols == l_ref[...]
        g_row = jnp.sum(jnp.where(m, x, 0.0), axis=1, keepdims=True)
        g = jnp.clip(g_row, EPS, 1.0 - EPS)
        s += (1.0 - SMOOTH) * jnp.sum(jnp.log(g) - jnp.log(1.0 - g))
        pa = jnp.clip(x[:HALF], EPS, 1.0 - EPS)
        pb = jnp.clip(x[HALF:], EPS, 1.0 - EPS)
        s += SMOOTH * jnp.sum(jnp.log(pa * pb))
        s += (1.0 - SMOOTH) * jnp.sum(jnp.log((1.0 - pa) * (1.0 - pb)))
    acc_ref[0, 0] += s

    @pl.when(step == GRID - 1)
    def _():
        o_ref[0, 0] = -acc_ref[0, 0] * (1.0 / (B * C))


def kernel(inputs, outputs, labels):
    del inputs
    lab2d = labels.astype(jnp.int32).reshape(B, 1)
    loss = pl.pallas_call(
        _dense_body,
        grid=(GRID,),
        in_specs=[
            pl.BlockSpec((STEP_ROWS, C), lambda i, k=k: (NSPLIT * i + k, 0))
            for k in range(NSPLIT)
        ] + [
            pl.BlockSpec((STEP_ROWS, 1), lambda i, k=k: (NSPLIT * i + k, 0))
            for k in range(NSPLIT)
        ],
        out_specs=pl.BlockSpec((1, 1), lambda i: (0, 0),
                               memory_space=pltpu.SMEM),
        out_shape=jax.ShapeDtypeStruct((1, 1), jnp.float32),
        scratch_shapes=[pltpu.SMEM((1, 1), jnp.float32)],
    )(*([outputs] * NSPLIT + [lab2d] * NSPLIT))
    return loss[0, 0]


# speedup vs baseline: 1.0055x; 1.0055x over previous
import functools

import jax
import jax.numpy as jnp
from jax import lax
from jax.experimental import pallas as pl
from jax.experimental.pallas import tpu as pltpu
from jax.experimental.pallas import tpu_sc as plsc

B = 16384
C = 1000
SMOOTH = 0.1
EPS = 1e-12

NSPLIT = 2
STEP_ROWS = 1024
GRID = B // (STEP_ROWS * NSPLIT)
HALF = STEP_ROWS // 2


def _dense_body(*refs):
    x_refs = refs[:NSPLIT]
    l_refs = refs[NSPLIT:2 * NSPLIT]
    o_ref, acc_ref = refs[2 * NSPLIT], refs[2 * NSPLIT + 1]
    step = pl.program_id(0)

    @pl.when(step == 0)
    def _():
        acc_ref[0, 0] = 0.0

    s = 0.0
    for x_ref, l_ref in zip(x_refs, l_refs):
        x = x_ref[...]
        cols = lax.broadcasted_iota(jnp.int32, (STEP_ROWS, C), 1)
        m = cols == l_ref[...]
        g_row = jnp.sum(jnp.where(m, x, 0.0), axis=1, keepdims=True)
        g = jnp.clip(g_row, EPS, 1.0 - EPS)
        s += (1.0 - SMOOTH) * jnp.sum(jnp.log(g) - jnp.log(1.0 - g))
        pa = jnp.clip(x[:HALF], EPS, 1.0 - EPS)
        pb = jnp.clip(x[HALF:], EPS, 1.0 - EPS)
        s += SMOOTH * jnp.sum(jnp.log(pa * pb))
        s += (1.0 - SMOOTH) * jnp.sum(jnp.log((1.0 - pa) * (1.0 - pb)))
    acc_ref[0, 0] += s

    @pl.when(step == GRID - 1)
    def _():
        o_ref[0, 0] = -acc_ref[0, 0] * (1.0 / (B * C))


def kernel(inputs, outputs, labels):
    del inputs
    lab2d = labels.astype(jnp.int32).reshape(B, 1)
    loss = pl.pallas_call(
        _dense_body,
        grid=(GRID,),
        in_specs=[
            pl.BlockSpec((STEP_ROWS, C), lambda i, k=k: (NSPLIT * i + k, 0))
            for k in range(NSPLIT)
        ] + [
            pl.BlockSpec((STEP_ROWS, 1), lambda i, k=k: (NSPLIT * i + k, 0))
            for k in range(NSPLIT)
        ],
        out_specs=pl.BlockSpec((1, 1), lambda i: (0, 0),
                               memory_space=pltpu.SMEM),
        out_shape=jax.ShapeDtypeStruct((1, 1), jnp.float32),
        scratch_shapes=[pltpu.SMEM((1, 1), jnp.float32)],
    )(*([outputs] * NSPLIT + [lab2d] * NSPLIT))
    return loss[0, 0]
